# baseline (speedup 1.0000x reference)
import functools

import jax
import jax.numpy as jnp
from jax.experimental import pallas as pl
from jax.experimental.pallas import tpu as pltpu

_PACK = 640


def _mlp_body(x_ref, m1_ref, b1_ref, m2_ref, b2_ref, o_ref):
    x = x_ref[...]
    h = jnp.maximum(
        jnp.dot(x, m1_ref[...], preferred_element_type=jnp.float32) + b1_ref[...],
        0.0,
    )
    o = jnp.dot(h, m2_ref[...], preferred_element_type=jnp.float32) + b2_ref[...]
    o_ref[...] = o.astype(o_ref.dtype)


@functools.partial(jax.jit, static_argnames=("block_rows",))
def _mlp_forward(x, w1_t, b1, w2_t, b2, *, block_rows=512):
    B, f_in = x.shape
    f_out = w2_t.shape[1]
    reps = _PACK // f_in

    x = x.astype(jnp.float32)
    b_pad = (B + reps - 1) // reps * reps
    if b_pad != B:
        x = jnp.pad(x, ((0, b_pad - B), (0, 0)))

    rows = b_pad // reps
    xp = x.reshape(rows, _PACK)

    eye = jnp.eye(reps, dtype=jnp.float32)
    m1 = jnp.kron(eye, w1_t.astype(jnp.float32))
    m2 = jnp.kron(eye, w2_t.astype(jnp.float32))
    b1t = jnp.tile(b1.astype(jnp.float32), reps).reshape(1, _PACK)
    b2t = jnp.tile(b2.astype(jnp.float32), reps).reshape(1, _PACK)

    tr = min(block_rows, rows)
    r_pad = (rows + tr - 1) // tr * tr
    if r_pad != rows:
        xp = jnp.pad(xp, ((0, r_pad - rows), (0, 0)))
    nr = r_pad // tr

    out = pl.pallas_call(
        _mlp_body,
        out_shape=jax.ShapeDtypeStruct((r_pad, _PACK), jnp.float32),
        grid_spec=pl.GridSpec(
            grid=(nr,),
            in_specs=[
                pl.BlockSpec((tr, _PACK), lambda i: (i, 0)),
                pl.BlockSpec((_PACK, _PACK), lambda i: (0, 0)),
                pl.BlockSpec((1, _PACK), lambda i: (0, 0)),
                pl.BlockSpec((_PACK, _PACK), lambda i: (0, 0)),
                pl.BlockSpec((1, _PACK), lambda i: (0, 0)),
            ],
            out_specs=pl.BlockSpec((tr, _PACK), lambda i: (i, 0)),
        ),
        compiler_params=pltpu.CompilerParams(
            dimension_semantics=("parallel",),
        ),
    )(xp, m1, b1t, m2, b2t)

    out = out[:rows].reshape(b_pad, f_in)[:B, :f_out]
    return out


def kernel(x, w1_t, b1, w2_t, b2):
    return _mlp_forward(x, w1_t, b1, w2_t, b2, block_rows=512)


# speedup vs baseline: 21.2401x; 21.2401x over previous
import functools

import jax
import jax.numpy as jnp
from jax.experimental import pallas as pl
from jax.experimental.pallas import tpu as pltpu


def _mlp_t_body(xt_ref, w1_ref, b1_ref, w2_ref, b2_ref, o_ref):
    xt = xt_ref[...]
    h = jnp.maximum(
        jnp.dot(w1_ref[...], xt, preferred_element_type=jnp.float32)
        + b1_ref[...],
        0.0,
    )
    o = jnp.dot(w2_ref[...], h, preferred_element_type=jnp.float32) + b2_ref[...]
    o_ref[...] = o.astype(o_ref.dtype)


@functools.partial(jax.jit, static_argnames=("block_lanes",))
def _mlp_forward(x, w1_t, b1, w2_t, b2, *, block_lanes=32768):
    B, f_in = x.shape
    f_out = w2_t.shape[1]

    xt = x.astype(jnp.float32).T
    w1 = w1_t.astype(jnp.float32).T
    w2 = w2_t.astype(jnp.float32).T
    b1c = b1.astype(jnp.float32).reshape(f_in, 1)
    b2c = b2.astype(jnp.float32).reshape(f_out, 1)

    bs = min(block_lanes, B)
    while B % bs:
        bs //= 2
    nb = B // bs

    ot = pl.pallas_call(
        _mlp_t_body,
        out_shape=jax.ShapeDtypeStruct((f_out, B), jnp.float32),
        grid_spec=pl.GridSpec(
            grid=(nb,),
            in_specs=[
                pl.BlockSpec((f_in, bs), lambda i: (0, i)),
                pl.BlockSpec((f_in, f_in), lambda i: (0, 0)),
                pl.BlockSpec((f_in, 1), lambda i: (0, 0)),
                pl.BlockSpec((f_out, f_in), lambda i: (0, 0)),
                pl.BlockSpec((f_out, 1), lambda i: (0, 0)),
            ],
            out_specs=pl.BlockSpec((f_out, bs), lambda i: (0, i)),
        ),
        compiler_params=pltpu.CompilerParams(
            dimension_semantics=("parallel",),
        ),
    )(xt, w1, b1c, w2, b2c)

    return ot.T


def kernel(x, w1_t, b1, w2_t, b2):
    return _mlp_forward(x, w1_t, b1, w2_t, b2, block_lanes=32768)


# speedup vs baseline: 24.3710x; 1.1474x over previous
import functools

import jax
import jax.numpy as jnp
from jax.experimental import pallas as pl
from jax.experimental.pallas import tpu as pltpu


def _mlp_t_body(xt_ref, w1_ref, b1_ref, w2_ref, b2_ref, o_ref):
    xt = xt_ref[...]
    h = jnp.maximum(
        jnp.dot(w1_ref[...], xt, preferred_element_type=jnp.float32)
        + b1_ref[...],
        0.0,
    )
    o = jnp.dot(w2_ref[...], h, preferred_element_type=jnp.float32) + b2_ref[...]
    o_ref[...] = o.astype(o_ref.dtype)


@functools.partial(jax.jit, static_argnames=("block_lanes",))
def _mlp_forward(x, w1_t, b1, w2_t, b2, *, block_lanes=32768):
    B, f_in = x.shape
    f_out = w2_t.shape[1]

    xt = x.astype(jnp.float32).T
    w1 = w1_t.astype(jnp.float32).T
    w2 = w2_t.astype(jnp.float32).T
    b1c = b1.astype(jnp.float32).reshape(f_in, 1)
    b2c = b2.astype(jnp.float32).reshape(f_out, 1)

    bs = min(block_lanes, B)
    while B % bs:
        bs //= 2
    nb = B // bs

    ot = pl.pallas_call(
        _mlp_t_body,
        out_shape=jax.ShapeDtypeStruct((f_out, B), jnp.float32),
        grid_spec=pl.GridSpec(
            grid=(nb,),
            in_specs=[
                pl.BlockSpec((f_in, bs), lambda i: (0, i)),
                pl.BlockSpec((f_in, f_in), lambda i: (0, 0)),
                pl.BlockSpec((f_in, 1), lambda i: (0, 0)),
                pl.BlockSpec((f_out, f_in), lambda i: (0, 0)),
                pl.BlockSpec((f_out, 1), lambda i: (0, 0)),
            ],
            out_specs=pl.BlockSpec((f_out, bs), lambda i: (0, i)),
        ),
        compiler_params=pltpu.CompilerParams(
            dimension_semantics=("parallel",),
        ),
    )(xt, w1, b1c, w2, b2c)

    return ot.T


def kernel(x, w1_t, b1, w2_t, b2):
    return _mlp_forward(x, w1_t, b1, w2_t, b2, block_lanes=65536)


# speedup vs baseline: 25.0950x; 1.0297x over previous
import functools

import jax
import jax.numpy as jnp
from jax.experimental import pallas as pl
from jax.experimental.pallas import tpu as pltpu


def _mlp_t_body(xt_ref, w1_ref, b1_ref, w2_ref, b2_ref, o_ref):
    xt = xt_ref[...]
    h = jnp.maximum(
        jnp.dot(w1_ref[...], xt, preferred_element_type=jnp.float32)
        + b1_ref[...],
        0.0,
    )
    o = jnp.dot(w2_ref[...], h, preferred_element_type=jnp.float32) + b2_ref[...]
    o_ref[...] = o.astype(o_ref.dtype)


@functools.partial(jax.jit, static_argnames=("block_lanes",))
def _mlp_forward(x, w1_t, b1, w2_t, b2, *, block_lanes=32768):
    B, f_in = x.shape
    f_out = w2_t.shape[1]

    xt = x.astype(jnp.float32).T
    w1 = w1_t.astype(jnp.float32).T
    w2 = w2_t.astype(jnp.float32).T
    b1c = b1.astype(jnp.float32).reshape(f_in, 1)
    b2c = b2.astype(jnp.float32).reshape(f_out, 1)

    bs = min(block_lanes, B)
    while B % bs:
        bs //= 2
    nb = B // bs

    ot = pl.pallas_call(
        _mlp_t_body,
        out_shape=jax.ShapeDtypeStruct((f_out, B), jnp.float32),
        grid_spec=pl.GridSpec(
            grid=(nb,),
            in_specs=[
                pl.BlockSpec((f_in, bs), lambda i: (0, i)),
                pl.BlockSpec((f_in, f_in), lambda i: (0, 0)),
                pl.BlockSpec((f_in, 1), lambda i: (0, 0)),
                pl.BlockSpec((f_out, f_in), lambda i: (0, 0)),
                pl.BlockSpec((f_out, 1), lambda i: (0, 0)),
            ],
            out_specs=pl.BlockSpec((f_out, bs), lambda i: (0, i)),
        ),
        compiler_params=pltpu.CompilerParams(
            dimension_semantics=("parallel",),
        ),
    )(xt, w1, b1c, w2, b2c)

    return ot.T


def kernel(x, w1_t, b1, w2_t, b2):
    return _mlp_forward(x, w1_t, b1, w2_t, b2, block_lanes=131072)


# speedup vs baseline: 27.9156x; 1.1124x over previous
import functools

import jax
import jax.numpy as jnp
from jax.experimental import pallas as pl
from jax.experimental.pallas import tpu as pltpu


_CONTRACT0 = (((0,), (0,)), ((), ()))


def _mlp_t_body(xt_ref, w1_ref, b1_ref, w2_ref, b2_ref, o_ref):
    xt = xt_ref[...]
    b1c = b1_ref[...].T
    b2c = b2_ref[...].T
    h = jnp.maximum(
        jax.lax.dot_general(w1_ref[...], xt, _CONTRACT0,
                            preferred_element_type=jnp.float32) + b1c,
        0.0,
    )
    o = jax.lax.dot_general(w2_ref[...], h, _CONTRACT0,
                            preferred_element_type=jnp.float32) + b2c
    o_ref[...] = o.astype(o_ref.dtype)


@functools.partial(jax.jit, static_argnames=("block_lanes",))
def _mlp_forward(x, w1_t, b1, w2_t, b2, *, block_lanes=32768):
    B, f_in = x.shape
    f_out = w2_t.shape[1]

    xt = x.astype(jnp.float32).T
    w1 = w1_t.astype(jnp.float32)
    w2 = w2_t.astype(jnp.float32)
    b1r = b1.astype(jnp.float32).reshape(1, f_in)
    b2r = b2.astype(jnp.float32).reshape(1, f_out)

    bs = min(block_lanes, B)
    while B % bs:
        bs //= 2
    nb = B // bs

    ot = pl.pallas_call(
        _mlp_t_body,
        out_shape=jax.ShapeDtypeStruct((f_out, B), jnp.float32),
        grid_spec=pl.GridSpec(
            grid=(nb,),
            in_specs=[
                pl.BlockSpec((f_in, bs), lambda i: (0, i)),
                pl.BlockSpec((f_in, f_in), lambda i: (0, 0)),
                pl.BlockSpec((1, f_in), lambda i: (0, 0)),
                pl.BlockSpec((f_in, f_out), lambda i: (0, 0)),
                pl.BlockSpec((1, f_out), lambda i: (0, 0)),
            ],
            out_specs=pl.BlockSpec((f_out, bs), lambda i: (0, i)),
        ),
        compiler_params=pltpu.CompilerParams(
            dimension_semantics=("parallel",),
        ),
    )(xt, w1, b1r, w2, b2r)

    return ot.T


def kernel(x, w1_t, b1, w2_t, b2):
    return _mlp_forward(x, w1_t, b1, w2_t, b2, block_lanes=131072)
